# baseline (speedup 1.0000x reference)
import jax
import jax.numpy as jnp
from jax.experimental import pallas as pl

_ROWS = 32
_V = 1000000
_SUB = 8
_LANES = _V // _SUB


def _body(x_ref, q_ref, p_ref, s_ref):
    x = x_ref[0]
    m = jnp.max(x)
    e = jnp.exp(x - m)
    s = jnp.sum(e)
    p = e / s
    p_ref[0] = p
    t = p / q_ref[0]
    mt = jnp.max(t)
    flat = (jax.lax.broadcasted_iota(jnp.int32, t.shape, 0) * _LANES
            + jax.lax.broadcasted_iota(jnp.int32, t.shape, 1))
    idx = jnp.min(jnp.where(t == mt, flat, _V))
    s_ref[0] = jnp.full((1, 128), idx, jnp.int32)


def kernel(logits):
    q = jax.random.exponential(jax.random.key(1), logits.shape,
                               dtype=jnp.float32)
    x3 = logits.reshape(_ROWS, _SUB, _LANES)
    q3 = q.reshape(_ROWS, _SUB, _LANES)
    row_spec = pl.BlockSpec((1, _SUB, _LANES), lambda i: (i, 0, 0))
    probs, samp = pl.pallas_call(
        _body,
        grid=(_ROWS,),
        in_specs=[row_spec, row_spec],
        out_specs=[row_spec,
                   pl.BlockSpec((1, 1, 128), lambda i: (i, 0, 0))],
        out_shape=[jax.ShapeDtypeStruct((_ROWS, _SUB, _LANES), jnp.float32),
                   jax.ShapeDtypeStruct((_ROWS, 1, 128), jnp.int32)],
    )(x3, q3)
    return probs.reshape(_ROWS, _V), samp[:, 0, 0]
